# baseline (speedup 1.0000x reference)
import functools

import jax
import jax.numpy as jnp
from jax import lax
from jax.experimental import pallas as pl
from jax.experimental.pallas import tpu as pltpu
from jax.experimental.pallas import tpu_sc as plsc

N = 10000
NP = 10240
E = 320000
D = 128
H = 128
G = 16

NC = 2
NS = 16
NW = NC * NS
ROWS_PER_TILE = NP // NS
CH = 128
CPW = 80
EPAD = NW * CPW * CH
NDB = 2
NIB = 4

RB = 1024
GRID = NP // RB


def _sc_segsum_body(h_hbm, src_hbm, dst_hbm, zeros_hbm, out_hbm,
                    src_v, dst_v, row, acc_sh):
    c = lax.axis_index("c")
    s = lax.axis_index("s")
    wid = c * NS + s
    own = pl.ds(s * ROWS_PER_TILE, ROWS_PER_TILE)
    pltpu.sync_copy(zeros_hbm, acc_sh.at[own])
    pltpu.sync_copy(src_hbm.at[wid], src_v)
    pltpu.sync_copy(dst_hbm.at[wid], dst_v)
    plsc.subcore_barrier()

    def chunk(j, carry):
        pltpu.sync_copy(h_hbm.at[src_v.at[j]], row)
        pltpu.sync_copy(row, acc_sh.at[dst_v.at[j]], add=True)
        return carry

    lax.fori_loop(0, CPW, chunk, 0)
    plsc.subcore_barrier()
    pltpu.sync_copy(acc_sh.at[own], out_hbm.at[c, own])


@functools.lru_cache(maxsize=1)
def _make_sc_segsum():
    return functools.partial(
        pl.kernel,
        out_type=jax.ShapeDtypeStruct((NC, NP, H), jnp.float32),
        mesh=plsc.VectorSubcoreMesh(core_axis_name="c", subcore_axis_name="s"),
        scratch_types=[
            pltpu.VMEM((CPW, CH), jnp.int32),
            pltpu.VMEM((CPW, CH), jnp.int32),
            pltpu.VMEM((CH, H), jnp.float32),
            pltpu.VMEM_SHARED((NP, H), jnp.float32),
        ],
    )(_sc_segsum_body)


def _sc_segsum(h, src3, dst3, zeros):
    return _make_sc_segsum()(h, src3, dst3, zeros)


def _row_mask(pid):
    rows = lax.broadcasted_iota(jnp.int32, (RB, 1), 0) + pid * RB
    return (rows < N).astype(jnp.float32)


def _mlp(z0, w1, b1, w2, b2):
    t = jnp.maximum(jnp.dot(z0, w1, preferred_element_type=jnp.float32) + b1, 0.0)
    return jnp.dot(t, w2, preferred_element_type=jnp.float32) + b2


def _store_stats(st_ref, z, pid):
    m = _row_mask(pid)
    zm = z * m
    s1 = jnp.sum(zm, axis=0, keepdims=True)
    s2 = jnp.sum(zm * z, axis=0, keepdims=True)
    st = jnp.concatenate([s1, s2, jnp.zeros((6, H), jnp.float32)], axis=0)

    @pl.when(pid == 0)
    def _():
        st_ref[...] = st

    @pl.when(pid != 0)
    def _():
        st_ref[...] = st_ref[...] + st


def _a0_body(p_ref, h_ref, w1_ref, b1_ref, w2_ref, b2_ref,
             gw1_ref, gb1_ref, gw2_ref, gb2_ref, eps_ref,
             z_ref, st_ref, g0_ref):
    pid = pl.program_id(0)
    z0 = p_ref[0] + p_ref[1] + h_ref[...]
    z = _mlp(z0, w1_ref[...], b1_ref[...], w2_ref[...], b2_ref[...])
    z_ref[...] = z
    _store_stats(st_ref, z, pid)
    sg = z0[:, 0:1] + eps_ref[:, 0:1] * h_ref[:, 0:1]
    tg = jnp.maximum(sg * gw1_ref[...] + gb1_ref[...], 0.0)
    g0_ref[...] = jnp.dot(tg, gw2_ref[...], preferred_element_type=jnp.float32) + gb2_ref[...]


def _a_body(p_ref, h_ref, w1_ref, b1_ref, w2_ref, b2_ref, z_ref, st_ref):
    pid = pl.program_id(0)
    z0 = p_ref[0] + p_ref[1] + h_ref[...]
    z = _mlp(z0, w1_ref[...], b1_ref[...], w2_ref[...], b2_ref[...])
    z_ref[...] = z
    _store_stats(st_ref, z, pid)


def _b_body(z_ref, st_ref, g_ref, b_ref, h_ref):
    mu = st_ref[0:1, :] * (1.0 / N)
    ex2 = st_ref[1:2, :] * (1.0 / N)
    var = ex2 - mu * mu
    scale = g_ref[...] * lax.rsqrt(var + 1e-5)
    h_ref[...] = z_ref[...] * scale + (b_ref[...] - mu * scale)


def _g1_body(p_ref, g_ref, w1_ref, b1_ref, w2_ref, b2_ref, eps_ref, o_ref):
    z0 = p_ref[0] + p_ref[1] + (1.0 + eps_ref[...]) * g_ref[...]
    o_ref[...] = _mlp(z0, w1_ref[...], b1_ref[...], w2_ref[...], b2_ref[...])


def _pool_body(h1_ref, h2_ref, h3_ref, g0_ref, g1_ref, bf_ref,
               pw1_ref, pb1_ref, pw2_ref, pb2_ref, out_ref, accf, accc):
    pid = pl.program_id(0)
    feat = jnp.concatenate([h1_ref[...], h2_ref[...], h3_ref[...],
                            g0_ref[...], g1_ref[...]], axis=1)
    ids = lax.broadcasted_iota(jnp.int32, (1, G), 1).astype(jnp.float32)
    oh = (bf_ref[...] == ids).astype(jnp.float32)
    dn = (((0,), (0,)), ((), ()))
    pf = lax.dot_general(oh, feat, dn, preferred_element_type=jnp.float32)
    pc = lax.dot_general(oh, jnp.ones((RB, H), jnp.float32), dn,
                         preferred_element_type=jnp.float32)

    @pl.when(pid == 0)
    def _():
        accf[...] = pf
        accc[...] = pc

    @pl.when(pid != 0)
    def _():
        accf[...] = accf[...] + pf
        accc[...] = accc[...] + pc

    @pl.when(pid == GRID - 1)
    def _():
        cnt = jnp.maximum(accc[...][:, 0:1], 1.0)
        pooled = accf[...] / cnt
        hdn = jnp.maximum(
            jnp.dot(pooled, pw1_ref[...], preferred_element_type=jnp.float32)
            + pb1_ref[...], 0.0)
        out_ref[...] = (jnp.dot(hdn, pw2_ref[...], preferred_element_type=jnp.float32)
                        + pb2_ref[...])


def _rows(shape):
    nd = len(shape)
    blk = (RB,) + shape[1:]
    return pl.BlockSpec(blk, lambda i: (i,) + (0,) * (nd - 1))


def _rows3(shape):
    blk = (shape[0], RB) + shape[2:]
    return pl.BlockSpec(blk, lambda i: (0, i) + (0,) * (len(shape) - 2))


def _full(shape):
    nd = len(shape)
    return pl.BlockSpec(shape, lambda i: (0,) * nd)


def _tc_call(body, in_specs, out_specs, out_shapes, scratch_shapes=()):
    return pl.pallas_call(
        body,
        grid=(GRID,),
        in_specs=in_specs,
        out_specs=out_specs,
        out_shape=out_shapes,
        scratch_shapes=list(scratch_shapes),
    )


_NPH = jax.ShapeDtypeStruct((NP, H), jnp.float32)
_ST = jax.ShapeDtypeStruct((8, H), jnp.float32)

_tc_a0 = _tc_call(
    _a0_body,
    [_rows3((NC, NP, H)), _rows((NP, H)), _full((H, H)), _full((1, H)),
     _full((H, H)), _full((1, H)), _full((1, H)), _full((1, H)),
     _full((H, H)), _full((1, H)), _full((1, H))],
    [_rows((NP, H)), _full((8, H)), _rows((NP, H))],
    [_NPH, _ST, _NPH],
)

_tc_a = _tc_call(
    _a_body,
    [_rows3((NC, NP, H)), _rows((NP, H)), _full((H, H)), _full((1, H)),
     _full((H, H)), _full((1, H))],
    [_rows((NP, H)), _full((8, H))],
    [_NPH, _ST],
)

_tc_b = _tc_call(
    _b_body,
    [_rows((NP, H)), _full((8, H)), _full((1, H)), _full((1, H))],
    _rows((NP, H)),
    _NPH,
)

_tc_g1 = _tc_call(
    _g1_body,
    [_rows3((NC, NP, H)), _rows((NP, H)), _full((H, H)), _full((1, H)),
     _full((H, H)), _full((1, H)), _full((1, H))],
    _rows((NP, H)),
    _NPH,
)

_PIN = 5 * H
_PH = 2 * H

_tc_pool = _tc_call(
    _pool_body,
    [_rows((NP, H))] * 5 + [_rows((NP, 1)), _full((_PIN, _PH)), _full((1, _PH)),
                            _full((_PH, D)), _full((1, D))],
    _full((G, D)),
    jax.ShapeDtypeStruct((G, D), jnp.float32),
    scratch_shapes=[pltpu.VMEM((G, _PIN), jnp.float32),
                    pltpu.VMEM((G, H), jnp.float32)],
)


def kernel(x, edge_index, batch, bW1, bb1, bW2, bb2, bng, bnb,
           g0W1, g0b1, g0W2, g0b2, g0eps,
           g1W1, g1b1, g1W2, g1b2, g1eps,
           pW1, pb1, pW2, pb2):
    xp = jnp.pad(x, ((0, NP - N), (0, 0)))
    src3 = jnp.concatenate([edge_index[0],
                            jnp.zeros((EPAD - E,), jnp.int32)]).reshape(
                                NW, CPW, CH)
    dst3 = jnp.concatenate([edge_index[1],
                            jnp.full((EPAD - E,), N, jnp.int32)]).reshape(
                                NW, CPW, CH)
    zeros = jnp.zeros((ROWS_PER_TILE, H), jnp.float32)
    bfp = jnp.pad(batch.astype(jnp.float32).reshape(N, 1),
                  ((0, NP - N), (0, 0)), constant_values=float(G))
    e0 = jnp.broadcast_to(jnp.reshape(g0eps, (1, 1)), (1, H))
    e1 = jnp.broadcast_to(jnp.reshape(g1eps, (1, 1)), (1, H))

    pX = _sc_segsum(xp, src3, dst3, zeros)
    z0, st0, gout0 = _tc_a0(pX, xp, bW1[0], bb1[0][None], bW2[0], bb2[0][None],
                            g0W1, g0b1[None], g0W2, g0b2[None], e0)
    h1 = _tc_b(z0, st0, bng[0][None], bnb[0][None])
    pg = _sc_segsum(gout0, src3, dst3, zeros)
    p1 = _sc_segsum(h1, src3, dst3, zeros)
    gout1 = _tc_g1(pg, gout0, g1W1, g1b1[None], g1W2, g1b2[None], e1)
    z1, st1 = _tc_a(p1, h1, bW1[1], bb1[1][None], bW2[1], bb2[1][None])
    h2 = _tc_b(z1, st1, bng[1][None], bnb[1][None])
    p2 = _sc_segsum(h2, src3, dst3, zeros)
    z2, st2 = _tc_a(p2, h2, bW1[2], bb1[2][None], bW2[2], bb2[2][None])
    h3 = _tc_b(z2, st2, bng[2][None], bnb[2][None])
    out = _tc_pool(h1, h2, h3, gout0, gout1, bfp,
                   pW1, pb1[None], pW2, pb2[None])
    return out


# speedup vs baseline: 2.4317x; 2.4317x over previous
import functools

import jax
import jax.numpy as jnp
from jax import lax
from jax.experimental import pallas as pl
from jax.experimental.pallas import tpu as pltpu
from jax.experimental.pallas import tpu_sc as plsc

N = 10000
NP = 10240
E = 320000
D = 128
H = 128
G = 16

NC = 2
NS = 16
NW = NC * NS
ROWS_PER_TILE = NP // NS
CH = 80
CPW = 125
NDB = 2
NIB = 4

RB = 1024
GRID = NP // RB


def _sc_segsum_body(h_hbm, src_hbm, dst_hbm, zeros_hbm, out_hbm,
                    src_v, dst_v, row, acc_sh):
    c = lax.axis_index("c")
    s = lax.axis_index("s")
    wid = c * NS + s
    own = pl.ds(s * ROWS_PER_TILE, ROWS_PER_TILE)
    pltpu.sync_copy(zeros_hbm, acc_sh.at[own])
    pltpu.sync_copy(src_hbm.at[wid], src_v)
    pltpu.sync_copy(dst_hbm.at[wid], dst_v)
    plsc.subcore_barrier()

    def chunk(j, carry):
        pltpu.sync_copy(h_hbm.at[src_v.at[j]], row)
        pltpu.sync_copy(row, acc_sh.at[dst_v.at[j]], add=True)
        return carry

    lax.fori_loop(0, CPW, chunk, 0)
    plsc.subcore_barrier()
    pltpu.sync_copy(acc_sh.at[own], out_hbm.at[c, own])


@functools.lru_cache(maxsize=1)
def _make_sc_segsum():
    return functools.partial(
        pl.kernel,
        out_type=jax.ShapeDtypeStruct((NC, NP, H), jnp.float32),
        mesh=plsc.VectorSubcoreMesh(core_axis_name="c", subcore_axis_name="s"),
        scratch_types=[
            pltpu.VMEM((CPW, CH), jnp.int32),
            pltpu.VMEM((CPW, CH), jnp.int32),
            pltpu.VMEM((CH, H), jnp.float32),
            pltpu.VMEM_SHARED((NP, H), jnp.float32),
        ],
    )(_sc_segsum_body)


def _sc_segsum(h, src3, dst3, zeros):
    return _make_sc_segsum()(h, src3, dst3, zeros)


def _row_mask(pid):
    rows = lax.broadcasted_iota(jnp.int32, (RB, 1), 0) + pid * RB
    return (rows < N).astype(jnp.float32)


def _mlp(z0, w1, b1, w2, b2):
    t = jnp.maximum(jnp.dot(z0, w1, preferred_element_type=jnp.float32) + b1, 0.0)
    return jnp.dot(t, w2, preferred_element_type=jnp.float32) + b2


def _store_stats(st_ref, z, pid):
    m = _row_mask(pid)
    zm = z * m
    s1 = jnp.sum(zm, axis=0, keepdims=True)
    s2 = jnp.sum(zm * z, axis=0, keepdims=True)
    st = jnp.concatenate([s1, s2, jnp.zeros((6, H), jnp.float32)], axis=0)

    @pl.when(pid == 0)
    def _():
        st_ref[...] = st

    @pl.when(pid != 0)
    def _():
        st_ref[...] = st_ref[...] + st


def _a0_body(p_ref, h_ref, w1_ref, b1_ref, w2_ref, b2_ref,
             gw1_ref, gb1_ref, gw2_ref, gb2_ref, eps_ref,
             z_ref, st_ref, g0_ref):
    pid = pl.program_id(0)
    z0 = p_ref[0] + p_ref[1] + h_ref[...]
    z = _mlp(z0, w1_ref[...], b1_ref[...], w2_ref[...], b2_ref[...])
    z_ref[...] = z
    _store_stats(st_ref, z, pid)
    sg = z0[:, 0:1] + eps_ref[:, 0:1] * h_ref[:, 0:1]
    tg = jnp.maximum(sg * gw1_ref[...] + gb1_ref[...], 0.0)
    g0_ref[...] = jnp.dot(tg, gw2_ref[...], preferred_element_type=jnp.float32) + gb2_ref[...]


def _a_body(p_ref, h_ref, w1_ref, b1_ref, w2_ref, b2_ref, z_ref, st_ref):
    pid = pl.program_id(0)
    z0 = p_ref[0] + p_ref[1] + h_ref[...]
    z = _mlp(z0, w1_ref[...], b1_ref[...], w2_ref[...], b2_ref[...])
    z_ref[...] = z
    _store_stats(st_ref, z, pid)


def _b_body(z_ref, st_ref, g_ref, b_ref, h_ref):
    mu = st_ref[0:1, :] * (1.0 / N)
    ex2 = st_ref[1:2, :] * (1.0 / N)
    var = ex2 - mu * mu
    scale = g_ref[...] * lax.rsqrt(var + 1e-5)
    h_ref[...] = z_ref[...] * scale + (b_ref[...] - mu * scale)


def _g1_body(p_ref, g_ref, w1_ref, b1_ref, w2_ref, b2_ref, eps_ref, o_ref):
    z0 = p_ref[0] + p_ref[1] + (1.0 + eps_ref[...]) * g_ref[...]
    o_ref[...] = _mlp(z0, w1_ref[...], b1_ref[...], w2_ref[...], b2_ref[...])


def _pool_body(h1_ref, h2_ref, h3_ref, g0_ref, g1_ref, bf_ref,
               pw1_ref, pb1_ref, pw2_ref, pb2_ref, out_ref, accf, accc):
    pid = pl.program_id(0)
    feat = jnp.concatenate([h1_ref[...], h2_ref[...], h3_ref[...],
                            g0_ref[...], g1_ref[...]], axis=1)
    ids = lax.broadcasted_iota(jnp.int32, (1, G), 1).astype(jnp.float32)
    oh = (bf_ref[...] == ids).astype(jnp.float32)
    dn = (((0,), (0,)), ((), ()))
    pf = lax.dot_general(oh, feat, dn, preferred_element_type=jnp.float32)
    pc = lax.dot_general(oh, jnp.ones((RB, H), jnp.float32), dn,
                         preferred_element_type=jnp.float32)

    @pl.when(pid == 0)
    def _():
        accf[...] = pf
        accc[...] = pc

    @pl.when(pid != 0)
    def _():
        accf[...] = accf[...] + pf
        accc[...] = accc[...] + pc

    @pl.when(pid == GRID - 1)
    def _():
        cnt = jnp.maximum(accc[...][:, 0:1], 1.0)
        pooled = accf[...] / cnt
        hdn = jnp.maximum(
            jnp.dot(pooled, pw1_ref[...], preferred_element_type=jnp.float32)
            + pb1_ref[...], 0.0)
        out_ref[...] = (jnp.dot(hdn, pw2_ref[...], preferred_element_type=jnp.float32)
                        + pb2_ref[...])


def _rows(shape):
    nd = len(shape)
    blk = (RB,) + shape[1:]
    return pl.BlockSpec(blk, lambda i: (i,) + (0,) * (nd - 1))


def _rows3(shape):
    blk = (shape[0], RB) + shape[2:]
    return pl.BlockSpec(blk, lambda i: (0, i) + (0,) * (len(shape) - 2))


def _full(shape):
    nd = len(shape)
    return pl.BlockSpec(shape, lambda i: (0,) * nd)


def _tc_call(body, in_specs, out_specs, out_shapes, scratch_shapes=()):
    return pl.pallas_call(
        body,
        grid=(GRID,),
        in_specs=in_specs,
        out_specs=out_specs,
        out_shape=out_shapes,
        scratch_shapes=list(scratch_shapes),
    )


_NPH = jax.ShapeDtypeStruct((NP, H), jnp.float32)
_ST = jax.ShapeDtypeStruct((8, H), jnp.float32)

_tc_a0 = _tc_call(
    _a0_body,
    [_rows3((NC, NP, H)), _rows((NP, H)), _full((H, H)), _full((1, H)),
     _full((H, H)), _full((1, H)), _full((1, H)), _full((1, H)),
     _full((H, H)), _full((1, H)), _full((1, H))],
    [_rows((NP, H)), _full((8, H)), _rows((NP, H))],
    [_NPH, _ST, _NPH],
)

_tc_a = _tc_call(
    _a_body,
    [_rows3((NC, NP, H)), _rows((NP, H)), _full((H, H)), _full((1, H)),
     _full((H, H)), _full((1, H))],
    [_rows((NP, H)), _full((8, H))],
    [_NPH, _ST],
)

_tc_b = _tc_call(
    _b_body,
    [_rows((NP, H)), _full((8, H)), _full((1, H)), _full((1, H))],
    _rows((NP, H)),
    _NPH,
)

_tc_g1 = _tc_call(
    _g1_body,
    [_rows3((NC, NP, H)), _rows((NP, H)), _full((H, H)), _full((1, H)),
     _full((H, H)), _full((1, H)), _full((1, H))],
    _rows((NP, H)),
    _NPH,
)

_PIN = 5 * H
_PH = 2 * H

_tc_pool = _tc_call(
    _pool_body,
    [_rows((NP, H))] * 5 + [_rows((NP, 1)), _full((_PIN, _PH)), _full((1, _PH)),
                            _full((_PH, D)), _full((1, D))],
    _full((G, D)),
    jax.ShapeDtypeStruct((G, D), jnp.float32),
    scratch_shapes=[pltpu.VMEM((G, _PIN), jnp.float32),
                    pltpu.VMEM((G, H), jnp.float32)],
)


def kernel(x, edge_index, batch, bW1, bb1, bW2, bb2, bng, bnb,
           g0W1, g0b1, g0W2, g0b2, g0eps,
           g1W1, g1b1, g1W2, g1b2, g1eps,
           pW1, pb1, pW2, pb2):
    xp = jnp.pad(x, ((0, NP - N), (0, 0)))
    src3 = edge_index[0].reshape(NW, CPW, CH)
    dst3 = edge_index[1].reshape(NW, CPW, CH)
    zeros = jnp.zeros((ROWS_PER_TILE, H), jnp.float32)
    bfp = jnp.pad(batch.astype(jnp.float32).reshape(N, 1),
                  ((0, NP - N), (0, 0)), constant_values=float(G))
    e0 = jnp.broadcast_to(jnp.reshape(g0eps, (1, 1)), (1, H))
    e1 = jnp.broadcast_to(jnp.reshape(g1eps, (1, 1)), (1, H))

    pX = _sc_segsum(xp, src3, dst3, zeros)
    z0, st0, gout0 = _tc_a0(pX, xp, bW1[0], bb1[0][None], bW2[0], bb2[0][None],
                            g0W1, g0b1[None], g0W2, g0b2[None], e0)
    h1 = _tc_b(z0, st0, bng[0][None], bnb[0][None])
    pg = _sc_segsum(gout0, src3, dst3, zeros)
    p1 = _sc_segsum(h1, src3, dst3, zeros)
    gout1 = _tc_g1(pg, gout0, g1W1, g1b1[None], g1W2, g1b2[None], e1)
    z1, st1 = _tc_a(p1, h1, bW1[1], bb1[1][None], bW2[1], bb2[1][None])
    h2 = _tc_b(z1, st1, bng[1][None], bnb[1][None])
    p2 = _sc_segsum(h2, src3, dst3, zeros)
    z2, st2 = _tc_a(p2, h2, bW1[2], bb1[2][None], bW2[2], bb2[2][None])
    h3 = _tc_b(z2, st2, bng[2][None], bnb[2][None])
    out = _tc_pool(h1, h2, h3, gout0, gout1, bfp,
                   pW1, pb1[None], pW2, pb2[None])
    return out


# speedup vs baseline: 4.0226x; 1.6543x over previous
import functools

import jax
import jax.numpy as jnp
from jax import lax
from jax.experimental import pallas as pl
from jax.experimental.pallas import tpu as pltpu
from jax.experimental.pallas import tpu_sc as plsc

N = 10000
NP = 10240
E = 320000
D = 128
H = 128
G = 16

NC = 2
NS = 16
NW = NC * NS
ROWS_PER_TILE = NP // NS
CH = 100
CPW = 100
NDB = 2
NIB = 4

RB = 1024
GRID = NP // RB


def _sc_segsum_body(h_hbm, idx_hbm, zeros_hbm, out_hbm,
                    i0, i1, i2, i3, r0, r1,
                    gs0, gs1, ss0, ss1, is0, is1, is2, is3, acc_sh):
    iring = (i0, i1, i2, i3)
    isem = (is0, is1, is2, is3)
    rows = (r0, r1)
    gsem = (gs0, gs1)
    ssem = (ss0, ss1)
    c = lax.axis_index("c")
    s = lax.axis_index("s")
    wid = c * NS + s
    own = pl.ds(s * ROWS_PER_TILE, ROWS_PER_TILE)
    pltpu.sync_copy(zeros_hbm, acc_sh.at[own])
    plsc.subcore_barrier()

    def fire_idx(k, j):
        pltpu.async_copy(idx_hbm.at[wid, j], iring[k], isem[k])

    def wait_idx(k, j):
        pltpu.make_async_copy(idx_hbm.at[wid, j], iring[k], isem[k]).wait()

    def fire_gather(b, k, j):
        pltpu.async_copy(h_hbm.at[iring[k].at[0]], rows[b], gsem[b])

    def wait_gather(b, k, j):
        pltpu.make_async_copy(h_hbm.at[iring[k].at[0]], rows[b],
                              gsem[b]).wait()

    def fire_scatter(b, k, j):
        pltpu.async_copy(rows[b], acc_sh.at[iring[k].at[1]], ssem[b],
                         add=True)

    def wait_scatter(b, k, j):
        pltpu.make_async_copy(rows[b], acc_sh.at[iring[k].at[1]],
                              ssem[b]).wait()

    for k in range(NIB):
        fire_idx(k, k)
    wait_idx(0, 0)
    fire_gather(0, 0, 0)
    wait_idx(1, 1)
    fire_gather(1, 1, 1)

    def group(g, carry):
        for b in range(4):
            j = 4 * g + b
            d = b & 1
            wait_gather(d, b, j)
            fire_scatter(d, b, j)

            @pl.when(j + 2 < CPW)
            def _():
                wait_scatter(d, b, j)
                @pl.when(j + NIB < CPW)
                def _():
                    fire_idx(b, j + NIB)

                wait_idx((b + 2) % NIB, j + 2)
                fire_gather(d, (b + 2) % NIB, j + 2)
        return carry

    lax.fori_loop(0, CPW // 4, group, 0)
    wait_scatter(0, 2, CPW - 2)
    wait_scatter(1, 3, CPW - 1)
    plsc.subcore_barrier()
    pltpu.sync_copy(acc_sh.at[own], out_hbm.at[c, own])


@functools.lru_cache(maxsize=1)
def _make_sc_segsum():
    return functools.partial(
        pl.kernel,
        out_type=jax.ShapeDtypeStruct((NC, NP, H), jnp.float32),
        mesh=plsc.VectorSubcoreMesh(core_axis_name="c", subcore_axis_name="s"),
        scratch_types=[pltpu.VMEM((2, CH), jnp.int32)] * NIB + [
            pltpu.VMEM((CH, H), jnp.float32),
            pltpu.VMEM((CH, H), jnp.float32),
        ] + [pltpu.SemaphoreType.DMA] * (2 * NDB + NIB)
        + [pltpu.VMEM_SHARED((NP, H), jnp.float32)],
    )(_sc_segsum_body)


def _sc_segsum(h, idx4, zeros):
    return _make_sc_segsum()(h, idx4, zeros)


def _row_mask(pid):
    rows = lax.broadcasted_iota(jnp.int32, (RB, 1), 0) + pid * RB
    return (rows < N).astype(jnp.float32)


def _mlp(z0, w1, b1, w2, b2):
    t = jnp.maximum(jnp.dot(z0, w1, preferred_element_type=jnp.float32) + b1, 0.0)
    return jnp.dot(t, w2, preferred_element_type=jnp.float32) + b2


def _store_stats(st_ref, z, pid):
    m = _row_mask(pid)
    zm = z * m
    s1 = jnp.sum(zm, axis=0, keepdims=True)
    s2 = jnp.sum(zm * z, axis=0, keepdims=True)
    st = jnp.concatenate([s1, s2, jnp.zeros((6, H), jnp.float32)], axis=0)

    @pl.when(pid == 0)
    def _():
        st_ref[...] = st

    @pl.when(pid != 0)
    def _():
        st_ref[...] = st_ref[...] + st


def _a0_body(p_ref, h_ref, w1_ref, b1_ref, w2_ref, b2_ref,
             gw1_ref, gb1_ref, gw2_ref, gb2_ref, eps_ref,
             z_ref, st_ref, g0_ref):
    pid = pl.program_id(0)
    z0 = p_ref[0] + p_ref[1] + h_ref[...]
    z = _mlp(z0, w1_ref[...], b1_ref[...], w2_ref[...], b2_ref[...])
    z_ref[...] = z
    _store_stats(st_ref, z, pid)
    sg = z0[:, 0:1] + eps_ref[:, 0:1] * h_ref[:, 0:1]
    tg = jnp.maximum(sg * gw1_ref[...] + gb1_ref[...], 0.0)
    g0_ref[...] = jnp.dot(tg, gw2_ref[...], preferred_element_type=jnp.float32) + gb2_ref[...]


def _a_body(p_ref, h_ref, w1_ref, b1_ref, w2_ref, b2_ref, z_ref, st_ref):
    pid = pl.program_id(0)
    z0 = p_ref[0] + p_ref[1] + h_ref[...]
    z = _mlp(z0, w1_ref[...], b1_ref[...], w2_ref[...], b2_ref[...])
    z_ref[...] = z
    _store_stats(st_ref, z, pid)


def _b_body(z_ref, st_ref, g_ref, b_ref, h_ref):
    mu = st_ref[0:1, :] * (1.0 / N)
    ex2 = st_ref[1:2, :] * (1.0 / N)
    var = ex2 - mu * mu
    scale = g_ref[...] * lax.rsqrt(var + 1e-5)
    h_ref[...] = z_ref[...] * scale + (b_ref[...] - mu * scale)


def _g1_body(p_ref, g_ref, w1_ref, b1_ref, w2_ref, b2_ref, eps_ref, o_ref):
    z0 = p_ref[0] + p_ref[1] + (1.0 + eps_ref[...]) * g_ref[...]
    o_ref[...] = _mlp(z0, w1_ref[...], b1_ref[...], w2_ref[...], b2_ref[...])


def _pool_body(h1_ref, h2_ref, h3_ref, g0_ref, g1_ref, bf_ref,
               pw1_ref, pb1_ref, pw2_ref, pb2_ref, out_ref, accf, accc):
    pid = pl.program_id(0)
    feat = jnp.concatenate([h1_ref[...], h2_ref[...], h3_ref[...],
                            g0_ref[...], g1_ref[...]], axis=1)
    ids = lax.broadcasted_iota(jnp.int32, (1, G), 1).astype(jnp.float32)
    oh = (bf_ref[...] == ids).astype(jnp.float32)
    dn = (((0,), (0,)), ((), ()))
    pf = lax.dot_general(oh, feat, dn, preferred_element_type=jnp.float32)
    pc = lax.dot_general(oh, jnp.ones((RB, H), jnp.float32), dn,
                         preferred_element_type=jnp.float32)

    @pl.when(pid == 0)
    def _():
        accf[...] = pf
        accc[...] = pc

    @pl.when(pid != 0)
    def _():
        accf[...] = accf[...] + pf
        accc[...] = accc[...] + pc

    @pl.when(pid == GRID - 1)
    def _():
        cnt = jnp.maximum(accc[...][:, 0:1], 1.0)
        pooled = accf[...] / cnt
        hdn = jnp.maximum(
            jnp.dot(pooled, pw1_ref[...], preferred_element_type=jnp.float32)
            + pb1_ref[...], 0.0)
        out_ref[...] = (jnp.dot(hdn, pw2_ref[...], preferred_element_type=jnp.float32)
                        + pb2_ref[...])


def _rows(shape):
    nd = len(shape)
    blk = (RB,) + shape[1:]
    return pl.BlockSpec(blk, lambda i: (i,) + (0,) * (nd - 1))


def _rows3(shape):
    blk = (shape[0], RB) + shape[2:]
    return pl.BlockSpec(blk, lambda i: (0, i) + (0,) * (len(shape) - 2))


def _full(shape):
    nd = len(shape)
    return pl.BlockSpec(shape, lambda i: (0,) * nd)


def _tc_call(body, in_specs, out_specs, out_shapes, scratch_shapes=()):
    return pl.pallas_call(
        body,
        grid=(GRID,),
        in_specs=in_specs,
        out_specs=out_specs,
        out_shape=out_shapes,
        scratch_shapes=list(scratch_shapes),
    )


_NPH = jax.ShapeDtypeStruct((NP, H), jnp.float32)
_ST = jax.ShapeDtypeStruct((8, H), jnp.float32)

_tc_a0 = _tc_call(
    _a0_body,
    [_rows3((NC, NP, H)), _rows((NP, H)), _full((H, H)), _full((1, H)),
     _full((H, H)), _full((1, H)), _full((1, H)), _full((1, H)),
     _full((H, H)), _full((1, H)), _full((1, H))],
    [_rows((NP, H)), _full((8, H)), _rows((NP, H))],
    [_NPH, _ST, _NPH],
)

_tc_a = _tc_call(
    _a_body,
    [_rows3((NC, NP, H)), _rows((NP, H)), _full((H, H)), _full((1, H)),
     _full((H, H)), _full((1, H))],
    [_rows((NP, H)), _full((8, H))],
    [_NPH, _ST],
)

_tc_b = _tc_call(
    _b_body,
    [_rows((NP, H)), _full((8, H)), _full((1, H)), _full((1, H))],
    _rows((NP, H)),
    _NPH,
)

_tc_g1 = _tc_call(
    _g1_body,
    [_rows3((NC, NP, H)), _rows((NP, H)), _full((H, H)), _full((1, H)),
     _full((H, H)), _full((1, H)), _full((1, H))],
    _rows((NP, H)),
    _NPH,
)

_PIN = 5 * H
_PH = 2 * H

_tc_pool = _tc_call(
    _pool_body,
    [_rows((NP, H))] * 5 + [_rows((NP, 1)), _full((_PIN, _PH)), _full((1, _PH)),
                            _full((_PH, D)), _full((1, D))],
    _full((G, D)),
    jax.ShapeDtypeStruct((G, D), jnp.float32),
    scratch_shapes=[pltpu.VMEM((G, _PIN), jnp.float32),
                    pltpu.VMEM((G, H), jnp.float32)],
)


def kernel(x, edge_index, batch, bW1, bb1, bW2, bb2, bng, bnb,
           g0W1, g0b1, g0W2, g0b2, g0eps,
           g1W1, g1b1, g1W2, g1b2, g1eps,
           pW1, pb1, pW2, pb2):
    xp = jnp.pad(x, ((0, NP - N), (0, 0)))
    idx4 = jnp.stack([edge_index[0].reshape(NW, CPW, CH),
                      edge_index[1].reshape(NW, CPW, CH)], axis=2)
    zeros = jnp.zeros((ROWS_PER_TILE, H), jnp.float32)
    bfp = jnp.pad(batch.astype(jnp.float32).reshape(N, 1),
                  ((0, NP - N), (0, 0)), constant_values=float(G))
    e0 = jnp.broadcast_to(jnp.reshape(g0eps, (1, 1)), (1, H))
    e1 = jnp.broadcast_to(jnp.reshape(g1eps, (1, 1)), (1, H))

    pX = _sc_segsum(xp, idx4, zeros)
    z0, st0, gout0 = _tc_a0(pX, xp, bW1[0], bb1[0][None], bW2[0], bb2[0][None],
                            g0W1, g0b1[None], g0W2, g0b2[None], e0)
    h1 = _tc_b(z0, st0, bng[0][None], bnb[0][None])
    pg = _sc_segsum(gout0, idx4, zeros)
    p1 = _sc_segsum(h1, idx4, zeros)
    gout1 = _tc_g1(pg, gout0, g1W1, g1b1[None], g1W2, g1b2[None], e1)
    z1, st1 = _tc_a(p1, h1, bW1[1], bb1[1][None], bW2[1], bb2[1][None])
    h2 = _tc_b(z1, st1, bng[1][None], bnb[1][None])
    p2 = _sc_segsum(h2, idx4, zeros)
    z2, st2 = _tc_a(p2, h2, bW1[2], bb1[2][None], bW2[2], bb2[2][None])
    h3 = _tc_b(z2, st2, bng[2][None], bnb[2][None])
    out = _tc_pool(h1, h2, h3, gout0, gout1, bfp,
                   pW1, pb1[None], pW2, pb2[None])
    return out


# speedup vs baseline: 4.1149x; 1.0229x over previous
import functools

import jax
import jax.numpy as jnp
from jax import lax
from jax.experimental import pallas as pl
from jax.experimental.pallas import tpu as pltpu
from jax.experimental.pallas import tpu_sc as plsc

N = 10000
NP = 10240
E = 320000
D = 128
H = 128
G = 16

NC = 2
NS = 16
NW = NC * NS
ROWS_PER_TILE = NP // NS
CH = 125
CPW = 80
NDB = 2
NIB = 4

RB = 1024
GRID = NP // RB


def _sc_segsum_body(h_hbm, idx_hbm, zeros_hbm, out_hbm,
                    i0, i1, i2, i3, r0, r1,
                    gs0, gs1, ss0, ss1, is0, is1, is2, is3, acc_sh):
    iring = (i0, i1, i2, i3)
    isem = (is0, is1, is2, is3)
    rows = (r0, r1)
    gsem = (gs0, gs1)
    ssem = (ss0, ss1)
    c = lax.axis_index("c")
    s = lax.axis_index("s")
    wid = c * NS + s
    own = pl.ds(s * ROWS_PER_TILE, ROWS_PER_TILE)
    pltpu.sync_copy(zeros_hbm, acc_sh.at[own])
    plsc.subcore_barrier()

    def fire_idx(k, j):
        pltpu.async_copy(idx_hbm.at[wid, j], iring[k], isem[k])

    def wait_idx(k, j):
        pltpu.make_async_copy(idx_hbm.at[wid, j], iring[k], isem[k]).wait()

    def fire_gather(b, k, j):
        pltpu.async_copy(h_hbm.at[iring[k].at[0]], rows[b], gsem[b])

    def wait_gather(b, k, j):
        pltpu.make_async_copy(h_hbm.at[iring[k].at[0]], rows[b],
                              gsem[b]).wait()

    def fire_scatter(b, k, j):
        pltpu.async_copy(rows[b], acc_sh.at[iring[k].at[1]], ssem[b],
                         add=True)

    def wait_scatter(b, k, j):
        pltpu.make_async_copy(rows[b], acc_sh.at[iring[k].at[1]],
                              ssem[b]).wait()

    for k in range(NIB):
        fire_idx(k, k)
    wait_idx(0, 0)
    fire_gather(0, 0, 0)
    wait_idx(1, 1)
    fire_gather(1, 1, 1)

    def group(g, carry):
        for b in range(4):
            j = 4 * g + b
            d = b & 1
            wait_gather(d, b, j)
            fire_scatter(d, b, j)

            @pl.when(j + 2 < CPW)
            def _():
                wait_scatter(d, b, j)
                @pl.when(j + NIB < CPW)
                def _():
                    fire_idx(b, j + NIB)

                wait_idx((b + 2) % NIB, j + 2)
                fire_gather(d, (b + 2) % NIB, j + 2)
        return carry

    lax.fori_loop(0, CPW // 4, group, 0)
    wait_scatter(0, 2, CPW - 2)
    wait_scatter(1, 3, CPW - 1)
    plsc.subcore_barrier()
    pltpu.sync_copy(acc_sh.at[own], out_hbm.at[c, own])


@functools.lru_cache(maxsize=1)
def _make_sc_segsum():
    return functools.partial(
        pl.kernel,
        out_type=jax.ShapeDtypeStruct((NC, NP, H), jnp.float32),
        mesh=plsc.VectorSubcoreMesh(core_axis_name="c", subcore_axis_name="s"),
        scratch_types=[pltpu.VMEM((2, CH), jnp.int32)] * NIB + [
            pltpu.VMEM((CH, H), jnp.float32),
            pltpu.VMEM((CH, H), jnp.float32),
        ] + [pltpu.SemaphoreType.DMA] * (2 * NDB + NIB)
        + [pltpu.VMEM_SHARED((NP, H), jnp.float32)],
    )(_sc_segsum_body)


def _sc_segsum(h, idx4, zeros):
    return _make_sc_segsum()(h, idx4, zeros)


def _row_mask(pid):
    rows = lax.broadcasted_iota(jnp.int32, (RB, 1), 0) + pid * RB
    return (rows < N).astype(jnp.float32)


def _mlp(z0, w1, b1, w2, b2):
    t = jnp.maximum(jnp.dot(z0, w1, preferred_element_type=jnp.float32) + b1, 0.0)
    return jnp.dot(t, w2, preferred_element_type=jnp.float32) + b2


def _store_stats(st_ref, z, pid):
    m = _row_mask(pid)
    zm = z * m
    s1 = jnp.sum(zm, axis=0, keepdims=True)
    s2 = jnp.sum(zm * z, axis=0, keepdims=True)
    st = jnp.concatenate([s1, s2, jnp.zeros((6, H), jnp.float32)], axis=0)

    @pl.when(pid == 0)
    def _():
        st_ref[...] = st

    @pl.when(pid != 0)
    def _():
        st_ref[...] = st_ref[...] + st


def _a0_body(p_ref, h_ref, w1_ref, b1_ref, w2_ref, b2_ref,
             gw1_ref, gb1_ref, gw2_ref, gb2_ref, eps_ref,
             z_ref, st_ref, g0_ref):
    pid = pl.program_id(0)
    z0 = p_ref[0] + p_ref[1] + h_ref[...]
    z = _mlp(z0, w1_ref[...], b1_ref[...], w2_ref[...], b2_ref[...])
    z_ref[...] = z
    _store_stats(st_ref, z, pid)
    sg = z0[:, 0:1] + eps_ref[:, 0:1] * h_ref[:, 0:1]
    tg = jnp.maximum(sg * gw1_ref[...] + gb1_ref[...], 0.0)
    g0_ref[...] = jnp.dot(tg, gw2_ref[...], preferred_element_type=jnp.float32) + gb2_ref[...]


def _a_body(p_ref, h_ref, w1_ref, b1_ref, w2_ref, b2_ref, z_ref, st_ref):
    pid = pl.program_id(0)
    z0 = p_ref[0] + p_ref[1] + h_ref[...]
    z = _mlp(z0, w1_ref[...], b1_ref[...], w2_ref[...], b2_ref[...])
    z_ref[...] = z
    _store_stats(st_ref, z, pid)


def _b_body(z_ref, st_ref, g_ref, b_ref, h_ref):
    mu = st_ref[0:1, :] * (1.0 / N)
    ex2 = st_ref[1:2, :] * (1.0 / N)
    var = ex2 - mu * mu
    scale = g_ref[...] * lax.rsqrt(var + 1e-5)
    h_ref[...] = z_ref[...] * scale + (b_ref[...] - mu * scale)


def _g1_body(p_ref, g_ref, w1_ref, b1_ref, w2_ref, b2_ref, eps_ref, o_ref):
    z0 = p_ref[0] + p_ref[1] + (1.0 + eps_ref[...]) * g_ref[...]
    o_ref[...] = _mlp(z0, w1_ref[...], b1_ref[...], w2_ref[...], b2_ref[...])


def _pool_body(h1_ref, h2_ref, h3_ref, g0_ref, g1_ref, bf_ref,
               pw1_ref, pb1_ref, pw2_ref, pb2_ref, out_ref, accf, accc):
    pid = pl.program_id(0)
    feat = jnp.concatenate([h1_ref[...], h2_ref[...], h3_ref[...],
                            g0_ref[...], g1_ref[...]], axis=1)
    ids = lax.broadcasted_iota(jnp.int32, (1, G), 1).astype(jnp.float32)
    oh = (bf_ref[...] == ids).astype(jnp.float32)
    dn = (((0,), (0,)), ((), ()))
    pf = lax.dot_general(oh, feat, dn, preferred_element_type=jnp.float32)
    pc = lax.dot_general(oh, jnp.ones((RB, H), jnp.float32), dn,
                         preferred_element_type=jnp.float32)

    @pl.when(pid == 0)
    def _():
        accf[...] = pf
        accc[...] = pc

    @pl.when(pid != 0)
    def _():
        accf[...] = accf[...] + pf
        accc[...] = accc[...] + pc

    @pl.when(pid == GRID - 1)
    def _():
        cnt = jnp.maximum(accc[...][:, 0:1], 1.0)
        pooled = accf[...] / cnt
        hdn = jnp.maximum(
            jnp.dot(pooled, pw1_ref[...], preferred_element_type=jnp.float32)
            + pb1_ref[...], 0.0)
        out_ref[...] = (jnp.dot(hdn, pw2_ref[...], preferred_element_type=jnp.float32)
                        + pb2_ref[...])


def _rows(shape):
    nd = len(shape)
    blk = (RB,) + shape[1:]
    return pl.BlockSpec(blk, lambda i: (i,) + (0,) * (nd - 1))


def _rows3(shape):
    blk = (shape[0], RB) + shape[2:]
    return pl.BlockSpec(blk, lambda i: (0, i) + (0,) * (len(shape) - 2))


def _full(shape):
    nd = len(shape)
    return pl.BlockSpec(shape, lambda i: (0,) * nd)


def _tc_call(body, in_specs, out_specs, out_shapes, scratch_shapes=()):
    return pl.pallas_call(
        body,
        grid=(GRID,),
        in_specs=in_specs,
        out_specs=out_specs,
        out_shape=out_shapes,
        scratch_shapes=list(scratch_shapes),
    )


_NPH = jax.ShapeDtypeStruct((NP, H), jnp.float32)
_ST = jax.ShapeDtypeStruct((8, H), jnp.float32)

_tc_a0 = _tc_call(
    _a0_body,
    [_rows3((NC, NP, H)), _rows((NP, H)), _full((H, H)), _full((1, H)),
     _full((H, H)), _full((1, H)), _full((1, H)), _full((1, H)),
     _full((H, H)), _full((1, H)), _full((1, H))],
    [_rows((NP, H)), _full((8, H)), _rows((NP, H))],
    [_NPH, _ST, _NPH],
)

_tc_a = _tc_call(
    _a_body,
    [_rows3((NC, NP, H)), _rows((NP, H)), _full((H, H)), _full((1, H)),
     _full((H, H)), _full((1, H))],
    [_rows((NP, H)), _full((8, H))],
    [_NPH, _ST],
)

_tc_b = _tc_call(
    _b_body,
    [_rows((NP, H)), _full((8, H)), _full((1, H)), _full((1, H))],
    _rows((NP, H)),
    _NPH,
)

_tc_g1 = _tc_call(
    _g1_body,
    [_rows3((NC, NP, H)), _rows((NP, H)), _full((H, H)), _full((1, H)),
     _full((H, H)), _full((1, H)), _full((1, H))],
    _rows((NP, H)),
    _NPH,
)

_PIN = 5 * H
_PH = 2 * H

_tc_pool = _tc_call(
    _pool_body,
    [_rows((NP, H))] * 5 + [_rows((NP, 1)), _full((_PIN, _PH)), _full((1, _PH)),
                            _full((_PH, D)), _full((1, D))],
    _full((G, D)),
    jax.ShapeDtypeStruct((G, D), jnp.float32),
    scratch_shapes=[pltpu.VMEM((G, _PIN), jnp.float32),
                    pltpu.VMEM((G, H), jnp.float32)],
)


def kernel(x, edge_index, batch, bW1, bb1, bW2, bb2, bng, bnb,
           g0W1, g0b1, g0W2, g0b2, g0eps,
           g1W1, g1b1, g1W2, g1b2, g1eps,
           pW1, pb1, pW2, pb2):
    xp = jnp.pad(x, ((0, NP - N), (0, 0)))
    idx4 = jnp.stack([edge_index[0].reshape(NW, CPW, CH),
                      edge_index[1].reshape(NW, CPW, CH)], axis=2)
    zeros = jnp.zeros((ROWS_PER_TILE, H), jnp.float32)
    bfp = jnp.pad(batch.astype(jnp.float32).reshape(N, 1),
                  ((0, NP - N), (0, 0)), constant_values=float(G))
    e0 = jnp.broadcast_to(jnp.reshape(g0eps, (1, 1)), (1, H))
    e1 = jnp.broadcast_to(jnp.reshape(g1eps, (1, 1)), (1, H))

    pX = _sc_segsum(xp, idx4, zeros)
    z0, st0, gout0 = _tc_a0(pX, xp, bW1[0], bb1[0][None], bW2[0], bb2[0][None],
                            g0W1, g0b1[None], g0W2, g0b2[None], e0)
    h1 = _tc_b(z0, st0, bng[0][None], bnb[0][None])
    pg = _sc_segsum(gout0, idx4, zeros)
    p1 = _sc_segsum(h1, idx4, zeros)
    gout1 = _tc_g1(pg, gout0, g1W1, g1b1[None], g1W2, g1b2[None], e1)
    z1, st1 = _tc_a(p1, h1, bW1[1], bb1[1][None], bW2[1], bb2[1][None])
    h2 = _tc_b(z1, st1, bng[1][None], bnb[1][None])
    p2 = _sc_segsum(h2, idx4, zeros)
    z2, st2 = _tc_a(p2, h2, bW1[2], bb1[2][None], bW2[2], bb2[2][None])
    h3 = _tc_b(z2, st2, bng[2][None], bnb[2][None])
    out = _tc_pool(h1, h2, h3, gout0, gout1, bfp,
                   pW1, pb1[None], pW2, pb2[None])
    return out
